# baseline (speedup 1.0000x reference)
import jax
import jax.numpy as jnp
from jax.experimental import pallas as pl
from jax.experimental.pallas import tpu as pltpu

_TOPK = 8
_BLOCK_T = 2048


def _router_body(x_ref, emb_ref, w_ref, b_ref, wout_ref, iout_ref):
    xb = x_ref[...] + emb_ref[...]
    logits_t = jax.lax.dot_general(
        w_ref[...], xb, (((0,), (1,)), ((), ())),
        preferred_element_type=jnp.float32,
    ) + b_ref[...]
    e, t = logits_t.shape
    k = jax.lax.bitcast_convert_type(logits_t, jnp.int32)
    work = k ^ jax.lax.shift_right_arithmetic(k, 31) & jnp.int32(0x7FFFFFFF)
    rev_iota = jnp.int32(e - 1) - jax.lax.broadcasted_iota(jnp.int32, (e, t), 0)
    neg_inf = jnp.int32(-(2**31))
    pick_v, pick_i = [], []
    for _ in range(_TOPK):
        m = jnp.max(work, axis=0, keepdims=True)
        eq = work == m
        ri = jnp.max(jnp.where(eq, rev_iota, jnp.int32(-1)), axis=0,
                     keepdims=True)
        pick_v.append(m)
        pick_i.append(jnp.int32(e - 1) - ri)
        work = jnp.where(rev_iota == ri, neg_inf, work)
    pv = jnp.concatenate(pick_v, axis=0)
    pi = jnp.concatenate(pick_i, axis=0)
    vk = pv ^ jax.lax.shift_right_arithmetic(pv, 31) & jnp.int32(0x7FFFFFFF)
    v = jax.lax.bitcast_convert_type(vk, jnp.float32)
    ev = jnp.exp(v - v[:1])
    w = ev / jnp.sum(ev, axis=0, keepdims=True)
    wout_ref[...] = w.T
    iout_ref[...] = pi.T


def kernel(x, scale_embeddings, W, b, scale_idx):
    batch, seq, d = x.shape
    e = W.shape[-1]
    n = batch * seq
    n_emb = scale_embeddings.shape[0]
    row = jnp.clip(scale_idx, 0, n_emb - 1)
    emb = jnp.where(
        scale_idx >= 0,
        jax.lax.dynamic_index_in_dim(scale_embeddings, row, 0, keepdims=False),
        jnp.zeros((d,), scale_embeddings.dtype),
    )
    x2 = x.reshape(n, d)
    wout, iout = pl.pallas_call(
        _router_body,
        grid=(n // _BLOCK_T,),
        compiler_params=pltpu.CompilerParams(
            dimension_semantics=("parallel",),
        ),
        in_specs=[
            pl.BlockSpec((_BLOCK_T, d), lambda i: (i, 0)),
            pl.BlockSpec((1, d), lambda i: (0, 0)),
            pl.BlockSpec((d, e), lambda i: (0, 0)),
            pl.BlockSpec((e, 1), lambda i: (0, 0)),
        ],
        out_specs=[
            pl.BlockSpec((_BLOCK_T, _TOPK), lambda i: (i, 0)),
            pl.BlockSpec((_BLOCK_T, _TOPK), lambda i: (i, 0)),
        ],
        out_shape=[
            jax.ShapeDtypeStruct((n, _TOPK), jnp.float32),
            jax.ShapeDtypeStruct((n, _TOPK), jnp.int32),
        ],
    )(x2, emb.reshape(1, d), W, b.reshape(e, 1))
    return wout.reshape(batch, seq, _TOPK), iout.reshape(batch, seq, _TOPK)


# speedup vs baseline: 1.0676x; 1.0676x over previous
import jax
import jax.numpy as jnp
from jax.experimental import pallas as pl
from jax.experimental.pallas import tpu as pltpu

_TOPK = 8
_BLOCK_T = 4096


def _router_body(x_ref, emb_ref, w_ref, b_ref, wout_ref, iout_ref):
    w = w_ref[...]
    bias = jax.lax.dot_general(
        w, emb_ref[...], (((0,), (1,)), ((), ())),
        preferred_element_type=jnp.float32,
    ) + b_ref[...]
    logits_t = jax.lax.dot_general(
        w, x_ref[...], (((0,), (1,)), ((), ())),
        preferred_element_type=jnp.float32,
    ) + bias
    e, t = logits_t.shape
    k = jax.lax.bitcast_convert_type(logits_t, jnp.int32)
    work = k ^ jax.lax.shift_right_arithmetic(k, 31) & jnp.int32(0x7FFFFFFF)
    rev_iota = jnp.int32(e - 1) - jax.lax.broadcasted_iota(jnp.int32, (e, t), 0)
    neg_inf = jnp.int32(-(2**31))
    pick_v, pick_i = [], []
    for _ in range(_TOPK):
        m = jnp.max(work, axis=0, keepdims=True)
        eq = work == m
        ri = jnp.max(jnp.where(eq, rev_iota, jnp.int32(-1)), axis=0,
                     keepdims=True)
        pick_v.append(m)
        pick_i.append(jnp.int32(e - 1) - ri)
        work = jnp.where(rev_iota == ri, neg_inf, work)
    pv = jnp.concatenate(pick_v, axis=0)
    pi = jnp.concatenate(pick_i, axis=0)
    vk = pv ^ jax.lax.shift_right_arithmetic(pv, 31) & jnp.int32(0x7FFFFFFF)
    v = jax.lax.bitcast_convert_type(vk, jnp.float32)
    ev = jnp.exp(v - v[:1])
    w = ev / jnp.sum(ev, axis=0, keepdims=True)
    wout_ref[...] = w.T
    iout_ref[...] = pi.T


def kernel(x, scale_embeddings, W, b, scale_idx):
    batch, seq, d = x.shape
    e = W.shape[-1]
    n = batch * seq
    n_emb = scale_embeddings.shape[0]
    row = jnp.clip(scale_idx, 0, n_emb - 1)
    emb = jnp.where(
        scale_idx >= 0,
        jax.lax.dynamic_index_in_dim(scale_embeddings, row, 0, keepdims=False),
        jnp.zeros((d,), scale_embeddings.dtype),
    )
    x2 = x.reshape(n, d)
    wout, iout = pl.pallas_call(
        _router_body,
        grid=(n // _BLOCK_T,),
        compiler_params=pltpu.CompilerParams(
            dimension_semantics=("parallel",),
        ),
        in_specs=[
            pl.BlockSpec((_BLOCK_T, d), lambda i: (i, 0)),
            pl.BlockSpec((1, d), lambda i: (0, 0)),
            pl.BlockSpec((d, e), lambda i: (0, 0)),
            pl.BlockSpec((e, 1), lambda i: (0, 0)),
        ],
        out_specs=[
            pl.BlockSpec((_BLOCK_T, _TOPK), lambda i: (i, 0)),
            pl.BlockSpec((_BLOCK_T, _TOPK), lambda i: (i, 0)),
        ],
        out_shape=[
            jax.ShapeDtypeStruct((n, _TOPK), jnp.float32),
            jax.ShapeDtypeStruct((n, _TOPK), jnp.int32),
        ],
    )(x2, emb.reshape(1, d), W, b.reshape(e, 1))
    return wout.reshape(batch, seq, _TOPK), iout.reshape(batch, seq, _TOPK)
